# Initial kernel scaffold; baseline (speedup 1.0000x reference)
import jax
import jax.numpy as jnp
from jax.experimental import pallas as pl


def kernel(x, edge_index, W1, b1, W2, b2):
    raise NotImplementedError("write your pallas kernel here")


# speedup vs baseline: 10.8152x; 10.8152x over previous
import functools

import jax
import jax.numpy as jnp
from jax import lax
from jax.experimental import pallas as pl
from jax.experimental.pallas import tpu as pltpu
from jax.experimental.pallas import tpu_sc as plsc

N = 10000
E = 160000
LATENT = 256
HIDDEN = 256
OUT = 128

NTILES = 16
PT = E // NTILES
CH = 80
NCH = PT // CH
PT2 = E // 32
CH2 = 40
NCH2 = PT2 // CH2
RB = 1000
RPT = 640
RPT_LAST = N - 15 * RPT


def _sc_mesh():
    return plsc.VectorSubcoreMesh(core_axis_name="c", subcore_axis_name="s")


def _copy_share(sid, src, dst):
    @pl.when(sid < 15)
    def _():
        off = pl.multiple_of(sid * RPT, 8)
        pltpu.sync_copy(src.at[pl.ds(off, RPT)], dst.at[pl.ds(off, RPT)])

    @pl.when(sid == 15)
    def _():
        pltpu.sync_copy(src.at[pl.ds(15 * RPT, RPT_LAST)],
                        dst.at[pl.ds(15 * RPT, RPT_LAST)])


DEGW = 128


@functools.partial(
    pl.kernel,
    out_type=jax.ShapeDtypeStruct((N, DEGW), jnp.float32),
    mesh=_sc_mesh(),
    scratch_types=[
        pltpu.VMEM((NCH, CH), jnp.int32),
        pltpu.VMEM((CH, DEGW), jnp.float32),
        pltpu.VMEM_SHARED((N, DEGW), jnp.float32),
    ],
)
def _deg_kernel(dst_hbm, ones_hbm, zeros_hbm, out_hbm, dstv, ones, acc):
    cid = lax.axis_index("c")
    sid = lax.axis_index("s")

    @pl.when(cid == 0)
    def _():
        pltpu.sync_copy(dst_hbm.at[sid], dstv)
        pltpu.sync_copy(ones_hbm, ones)
        _copy_share(sid, zeros_hbm, acc)
        plsc.subcore_barrier()

        def count(j, carry):
            pltpu.sync_copy(ones, acc.at[dstv.at[j]], add=True)
            return carry

        lax.fori_loop(0, NCH, count, 0)
        plsc.subcore_barrier()
        _copy_share(sid, acc, out_hbm)


@functools.partial(
    pl.kernel,
    out_type=jax.ShapeDtypeStruct((2, N, HIDDEN // 2), jnp.float32),
    mesh=_sc_mesh(),
    scratch_types=[
        pltpu.VMEM((NCH, CH), jnp.int32),
        pltpu.VMEM((NCH, CH), jnp.int32),
        pltpu.VMEM((CH, HIDDEN // 2), jnp.float32),
        pltpu.VMEM_SHARED((N, HIDDEN // 2), jnp.float32),
        pltpu.SemaphoreType.DMA,
    ],
)
def _agg1_kernel(y_hbm, src_hbm, dst_hbm, zeros_hbm, out_hbm,
                 srcv, dstv, rows, acc, sem):
    cid = lax.axis_index("c")
    sid = lax.axis_index("s")
    pltpu.sync_copy(src_hbm.at[sid], srcv)
    pltpu.sync_copy(dst_hbm.at[sid], dstv)
    _copy_share(sid, zeros_hbm, acc)
    plsc.subcore_barrier()

    def chunk(j, carry):
        pltpu.async_copy(y_hbm.at[cid].at[srcv.at[j]], rows, sem).wait()
        pltpu.sync_copy(rows, acc.at[dstv.at[j]], add=True)
        return carry

    lax.fori_loop(0, NCH, chunk, 0)
    plsc.subcore_barrier()
    _copy_share(sid, acc, out_hbm.at[cid])


@functools.partial(
    pl.kernel,
    out_type=jax.ShapeDtypeStruct((2, N, OUT), jnp.float32),
    mesh=_sc_mesh(),
    scratch_types=[
        pltpu.VMEM((NCH2, CH2), jnp.int32),
        pltpu.VMEM((NCH2, CH2), jnp.int32),
        pltpu.VMEM((CH2, OUT), jnp.float32),
        pltpu.VMEM_SHARED((N, OUT), jnp.float32),
        pltpu.SemaphoreType.DMA,
    ],
)
def _agg2_kernel(y_hbm, src_hbm, dst_hbm, zeros_hbm, out_hbm,
                 srcv, dstv, rows, acc, sem):
    cid = lax.axis_index("c")
    sid = lax.axis_index("s")
    w = cid * NTILES + sid
    pltpu.sync_copy(src_hbm.at[w], srcv)
    pltpu.sync_copy(dst_hbm.at[w], dstv)
    _copy_share(sid, zeros_hbm, acc)
    plsc.subcore_barrier()

    def chunk(j, carry):
        pltpu.async_copy(y_hbm.at[srcv.at[j]], rows, sem).wait()
        pltpu.sync_copy(rows, acc.at[dstv.at[j]], add=True)
        return carry

    lax.fori_loop(0, NCH2, chunk, 0)
    plsc.subcore_barrier()
    _copy_share(sid, acc, out_hbm.at[cid])


def _b_body(x_ref, w_ref, deg_ref, y_ref, dinv_ref):
    dcol = lax.rsqrt(deg_ref[:, 0:1] + 1.0)
    dinv_ref[...] = dcol
    y = dcol * jnp.dot(x_ref[...], w_ref[...],
                       preferred_element_type=jnp.float32)
    y_ref[0] = y[:, :HIDDEN // 2]
    y_ref[1] = y[:, HIDDEN // 2:]


def _d_body(agg_ref, y1_ref, dinv_ref, b1_ref, w2_ref, y2_ref):
    dinv = dinv_ref[...]
    s = jnp.concatenate([agg_ref[0] + y1_ref[0], agg_ref[1] + y1_ref[1]],
                        axis=1)
    h = jnp.maximum(dinv * s + b1_ref[...], 0.0)
    y2_ref[...] = dinv * jnp.dot(h, w2_ref[...],
                                 preferred_element_type=jnp.float32)


def _f_body(agg_ref, y2_ref, dinv_ref, b2_ref, out_ref):
    s = agg_ref[0] + agg_ref[1] + y2_ref[...]
    out_ref[...] = dinv_ref[...] * s + b2_ref[...]


def _tc_b(x, w1, degp):
    return pl.pallas_call(
        _b_body,
        grid=(N // RB,),
        in_specs=[
            pl.BlockSpec((RB, LATENT), lambda i: (i, 0)),
            pl.BlockSpec((LATENT, HIDDEN), lambda i: (0, 0)),
            pl.BlockSpec((RB, DEGW), lambda i: (i, 0)),
        ],
        out_specs=[
            pl.BlockSpec((2, RB, HIDDEN // 2), lambda i: (0, i, 0)),
            pl.BlockSpec((RB, 1), lambda i: (i, 0)),
        ],
        out_shape=[
            jax.ShapeDtypeStruct((2, N, HIDDEN // 2), jnp.float32),
            jax.ShapeDtypeStruct((N, 1), jnp.float32),
        ],
    )(x, w1, degp)


def _tc_d(agg1, y1, dinv, b1, w2):
    return pl.pallas_call(
        _d_body,
        grid=(N // RB,),
        in_specs=[
            pl.BlockSpec((2, RB, HIDDEN // 2), lambda i: (0, i, 0)),
            pl.BlockSpec((2, RB, HIDDEN // 2), lambda i: (0, i, 0)),
            pl.BlockSpec((RB, 1), lambda i: (i, 0)),
            pl.BlockSpec((1, HIDDEN), lambda i: (0, 0)),
            pl.BlockSpec((HIDDEN, OUT), lambda i: (0, 0)),
        ],
        out_specs=pl.BlockSpec((RB, OUT), lambda i: (i, 0)),
        out_shape=jax.ShapeDtypeStruct((N, OUT), jnp.float32),
    )(agg1, y1, dinv, b1, w2)


def _tc_f(agg2, y2, dinv, b2):
    return pl.pallas_call(
        _f_body,
        grid=(N // RB,),
        in_specs=[
            pl.BlockSpec((2, RB, OUT), lambda i: (0, i, 0)),
            pl.BlockSpec((RB, OUT), lambda i: (i, 0)),
            pl.BlockSpec((RB, 1), lambda i: (i, 0)),
            pl.BlockSpec((1, OUT), lambda i: (0, 0)),
        ],
        out_specs=pl.BlockSpec((RB, OUT), lambda i: (i, 0)),
        out_shape=jax.ShapeDtypeStruct((N, OUT), jnp.float32),
    )(agg2, y2, dinv, b2)


def kernel(x, edge_index, W1, b1, W2, b2):
    src = edge_index[0].astype(jnp.int32)
    dst = edge_index[1].astype(jnp.int32)
    src1 = src.reshape(NTILES, NCH, CH)
    dst1 = dst.reshape(NTILES, NCH, CH)
    src2 = src.reshape(32, NCH2, CH2)
    dst2 = dst.reshape(32, NCH2, CH2)

    zeros_h = jnp.zeros((N, HIDDEN // 2), jnp.float32)
    zeros_o = jnp.zeros((N, OUT), jnp.float32)
    zeros_d = jnp.zeros((N, DEGW), jnp.float32)
    ones_d = jnp.ones((CH, DEGW), jnp.float32)

    degp = _deg_kernel(dst1, ones_d, zeros_d)
    y1, dinv = _tc_b(x, W1, degp)
    agg1 = _agg1_kernel(y1, src1, dst1, zeros_h)
    y2 = _tc_d(agg1, y1, dinv, b1.reshape(1, HIDDEN), W2)
    agg2 = _agg2_kernel(y2, src2, dst2, zeros_o)
    return _tc_f(agg2, y2, dinv, b2.reshape(1, OUT))
